# Initial kernel scaffold; baseline (speedup 1.0000x reference)
import jax
import jax.numpy as jnp
from jax.experimental import pallas as pl


def kernel(x0, x1, x2, edge_index, edge_type, P0, b0, P1, b1, P2, b2, basis1, comb1, Q1, K1, basis2, comb2, Q2, K2, Wi, bi):
    raise NotImplementedError("write your pallas kernel here")


# speedup vs baseline: 1.0025x; 1.0025x over previous
import jax
import jax.numpy as jnp
from jax.experimental import pallas as pl
from jax.experimental.pallas import tpu as pltpu

NPV = 16384
N = 3 * NPV
E = 786432
R = 4
H = 4
O = 16
HO = 64
NL = 5


def _proj_body(x0_ref, x1_ref, x2_ref, P0_ref, b0_ref, P1_ref, b1_ref,
               P2_ref, b2_ref, h0_ref, h1_ref, h2_ref):
    h0_ref[...] = jnp.maximum(
        jnp.dot(x0_ref[...], P0_ref[...], preferred_element_type=jnp.float32)
        + b0_ref[...], 0.0)
    h1_ref[...] = jnp.maximum(
        jnp.dot(x1_ref[...], P1_ref[...], preferred_element_type=jnp.float32)
        + b1_ref[...], 0.0)
    h2_ref[...] = jnp.maximum(
        jnp.dot(x2_ref[...], P2_ref[...], preferred_element_type=jnp.float32)
        + b2_ref[...], 0.0)


def _project(x0, x1, x2, P0, b0, P1, b1, P2, b2):
    BN = 2048
    grid = (NPV // BN,)
    out = pl.pallas_call(
        _proj_body,
        grid=grid,
        in_specs=[
            pl.BlockSpec((BN, 512), lambda i: (i, 0)),
            pl.BlockSpec((BN, 256), lambda i: (i, 0)),
            pl.BlockSpec((BN, 128), lambda i: (i, 0)),
            pl.BlockSpec((512, HO), lambda i: (0, 0)),
            pl.BlockSpec((HO,), lambda i: (0,)),
            pl.BlockSpec((256, HO), lambda i: (0, 0)),
            pl.BlockSpec((HO,), lambda i: (0,)),
            pl.BlockSpec((128, HO), lambda i: (0, 0)),
            pl.BlockSpec((HO,), lambda i: (0,)),
        ],
        out_specs=[
            pl.BlockSpec((BN, HO), lambda i: (i, 0)),
            pl.BlockSpec((BN, HO), lambda i: (i, 0)),
            pl.BlockSpec((BN, HO), lambda i: (i, 0)),
        ],
        out_shape=[jax.ShapeDtypeStruct((NPV, HO), jnp.float32)] * 3,
    )(x0, x1, x2, P0, b0, P1, b1, P2, b2)
    return jnp.concatenate(out, axis=0)


def _rgat_layer(x, src, dst, etype, basis, comb, Q, K):
    Wr = jnp.einsum('rb,bio->rio', comb, basis)
    xr = jnp.einsum('ni,rio->rno', x, Wr)
    xj = xr[etype, src].reshape(-1, H, O)
    xi = xr[etype, dst].reshape(-1, H, O)
    q = jnp.sum(xi * Q[etype], axis=-1)
    k = jnp.sum(xj * K[etype], axis=-1)
    alpha = jax.nn.leaky_relu(q + k, negative_slope=0.2)
    amax = jax.ops.segment_max(alpha, dst, num_segments=N)
    ex = jnp.exp(alpha - amax[dst])
    denom = jax.ops.segment_sum(ex, dst, num_segments=N)
    attn = ex / (denom[dst] + 1e-16)
    deg = jax.ops.segment_sum(jnp.ones_like(ex[:, :1]), dst, num_segments=N)
    attn = attn * deg[dst]
    msg = (attn[:, :, None] * xj).reshape(-1, H * O)
    return jax.ops.segment_sum(msg, dst, num_segments=N)


def kernel(x0, x1, x2, edge_index, edge_type, P0, b0, P1, b1, P2, b2,
           basis1, comb1, Q1, K1, basis2, comb2, Q2, K2, Wi, bi):
    src = edge_index[0]
    dst = edge_index[1]
    x = _project(x0, x1, x2, P0, b0, P1, b1, P2, b2)
    x = _rgat_layer(x, src, dst, edge_type, basis1, comb1, Q1, K1)
    x = _rgat_layer(x, src, dst, edge_type, basis2, comb2, Q2, K2)
    feat = jnp.concatenate([x[:NPV], x[NPV:2 * NPV], x[2 * NPV:]], axis=-1)
    return feat @ Wi + bi


# speedup vs baseline: 27.5466x; 27.4768x over previous
import functools

import jax
import jax.numpy as jnp
from jax import lax
from jax.experimental import pallas as pl
from jax.experimental.pallas import tpu as pltpu
from jax.experimental.pallas import tpu_sc as plsc

NPV = 16384
N = 3 * NPV
E = 786432
R = 4
H = 4
O = 16
HO = 64
NL = 5

NCORE = 2
NSUB = 16
NW = NCORE * NSUB
LANES = 16

C1 = 1024
C1J = C1 // 128
EW1 = E // NW
C2 = 1024
C2J = C2 // 128
EW2 = E // NSUB
STRIPE = N // NSUB

_mesh = plsc.VectorSubcoreMesh(core_axis_name="c", subcore_axis_name="s")


def _proj_body(x0_ref, x1_ref, x2_ref, P0_ref, b0_ref, P1_ref, b1_ref,
               P2_ref, b2_ref, out_ref):
    out_ref[0] = jnp.maximum(
        jnp.dot(x0_ref[...], P0_ref[...], preferred_element_type=jnp.float32)
        + b0_ref[...], 0.0)
    out_ref[1] = jnp.maximum(
        jnp.dot(x1_ref[...], P1_ref[...], preferred_element_type=jnp.float32)
        + b1_ref[...], 0.0)
    out_ref[2] = jnp.maximum(
        jnp.dot(x2_ref[...], P2_ref[...], preferred_element_type=jnp.float32)
        + b2_ref[...], 0.0)


def _project(x0, x1, x2, P0, b0, P1, b1, P2, b2):
    BN = 2048
    out = pl.pallas_call(
        _proj_body,
        grid=(NPV // BN,),
        in_specs=[
            pl.BlockSpec((BN, 512), lambda i: (i, 0)),
            pl.BlockSpec((BN, 256), lambda i: (i, 0)),
            pl.BlockSpec((BN, 128), lambda i: (i, 0)),
            pl.BlockSpec((512, HO), lambda i: (0, 0)),
            pl.BlockSpec((HO,), lambda i: (0,)),
            pl.BlockSpec((256, HO), lambda i: (0, 0)),
            pl.BlockSpec((HO,), lambda i: (0,)),
            pl.BlockSpec((128, HO), lambda i: (0, 0)),
            pl.BlockSpec((HO,), lambda i: (0,)),
        ],
        out_specs=pl.BlockSpec((3, BN, HO), lambda i: (0, i, 0)),
        out_shape=jax.ShapeDtypeStruct((3, NPV, HO), jnp.float32),
    )(x0, x1, x2, P0, b0, P1, b1, P2, b2)
    return out.reshape(N, HO)


def _xform_common(x_blk, basis_ref, comb_ref, Q_ref, K_ref, xr_ref, qk_ref):
    bn = x_blk.shape[0]
    for r in range(R):
        W_r = (comb_ref[r, 0] * basis_ref[0]
               + comb_ref[r, 1] * basis_ref[1])
        xr_r = jnp.dot(x_blk, W_r, preferred_element_type=jnp.float32)
        for h in range(H):
            xr_ref[h, r] = xr_r[:, h * O:(h + 1) * O]
        cols = []
        for h in range(H):
            qrow = Q_ref[r, h][None, :]
            cols.append(jnp.sum(xr_r[:, h * O:(h + 1) * O] * qrow,
                                axis=1, keepdims=True))
        for h in range(H):
            krow = K_ref[r, h][None, :]
            cols.append(jnp.sum(xr_r[:, h * O:(h + 1) * O] * krow,
                                axis=1, keepdims=True))
        cols.append(jnp.zeros((bn, 8), jnp.float32))
        qk_ref[r] = jnp.concatenate(cols, axis=1)


def _xform1_body(x_ref, basis_ref, comb_ref, Q_ref, K_ref, xr_ref, qk_ref):
    _xform_common(x_ref[...], basis_ref, comb_ref, Q_ref, K_ref,
                  xr_ref, qk_ref)


def _norm_x(s_refs, dlo_ref, dhi_ref):
    d = dlo_ref[...] + dhi_ref[...]
    deg = d[:, 4:5]
    parts = []
    for h in range(H):
        scale = deg / (d[:, h:h + 1] + 1e-16)
        parts.append(s_refs[h][...] * scale)
    return jnp.concatenate(parts, axis=1)


def _xform2_body(s0, s1, s2, s3, dlo_ref, dhi_ref, basis_ref, comb_ref,
                 Q_ref, K_ref, xr_ref, qk_ref):
    x_blk = _norm_x((s0, s1, s2, s3), dlo_ref, dhi_ref)
    _xform_common(x_blk, basis_ref, comb_ref, Q_ref, K_ref, xr_ref, qk_ref)


_XFORM_OUT = [
    jax.ShapeDtypeStruct((H, R, N, 16), jnp.float32),
    jax.ShapeDtypeStruct((R, N, 16), jnp.float32),
]

_W_SPECS = [
    pl.BlockSpec((2, HO, HO), lambda i: (0, 0, 0)),
    pl.BlockSpec((R, 2), lambda i: (0, 0)),
    pl.BlockSpec((R, H, O), lambda i: (0, 0, 0)),
    pl.BlockSpec((R, H, O), lambda i: (0, 0, 0)),
]

_BT = 2048


def _xform_out_specs():
    return [
        pl.BlockSpec((H, R, _BT, 16), lambda i: (0, 0, i, 0)),
        pl.BlockSpec((R, _BT, 16), lambda i: (0, i, 0)),
    ]


def _transform1(x, basis, comb, Q, K):
    xr, qk = pl.pallas_call(
        _xform1_body,
        grid=(N // _BT,),
        in_specs=[pl.BlockSpec((_BT, HO), lambda i: (i, 0))] + _W_SPECS,
        out_specs=_xform_out_specs(),
        out_shape=_XFORM_OUT,
    )(x, basis, comb, Q, K)
    return xr.reshape(H * R * N, 16), qk.reshape(R * N, 16)


def _s_spec(h, kn):
    return pl.BlockSpec((_BT, 16), lambda i, _o=(h % 2) * kn: (i + _o, 0))


def _transform2(SA, SB, dp, basis, comb, Q, K):
    kn = N // _BT
    xr, qk = pl.pallas_call(
        _xform2_body,
        grid=(kn,),
        in_specs=[
            _s_spec(0, kn), _s_spec(1, kn), _s_spec(2, kn), _s_spec(3, kn),
            pl.BlockSpec((_BT, 16), lambda i: (i, 0)),
            pl.BlockSpec((_BT, 16), lambda i, _k=kn: (i + _k, 0)),
        ] + _W_SPECS,
        out_specs=_xform_out_specs(),
        out_shape=_XFORM_OUT,
    )(SA, SA, SB, SB, dp, dp, basis, comb, Q, K)
    return xr.reshape(H * R * N, 16), qk.reshape(R * N, 16)


_GDN = lax.GatherDimensionNumbers(
    offset_dims=(), collapsed_slice_dims=(0,), start_index_map=(0,))


def _splat(vec, sel):
    return lax.gather(vec, sel, _GDN, slice_sizes=(1,),
                      mode=lax.GatherScatterMode.PROMISE_IN_BOUNDS)


def _p1_body(qk_hbm, src_hbm, dst_hbm, et_hbm, ex_hbm, dp_hbm,
             srcv, dstv, etv, idxq, idxk, qrows, krows, exrow, bufx,
             dspm, sem):
    cid = lax.axis_index("c")
    sid = lax.axis_index("s")
    wid = sid * NCORE + cid
    lane = lax.iota(jnp.int32, LANES)
    degc = jnp.where(lane == 4, 1.0, 0.0).astype(jnp.float32)
    m4 = lane < 4
    m8 = lane < 8
    low8 = jnp.maximum(lane - 8, 0).reshape(LANES, 1)
    sel4 = jnp.minimum(lane + 4, LANES - 1).reshape(LANES, 1)

    @pl.loop(0, C1)
    def _zero(i):
        exrow[i] = jnp.zeros((LANES,), jnp.float32)

    r0 = pl.multiple_of(sid * STRIPE, STRIPE)
    for t in range(STRIPE // C1):
        pltpu.sync_copy(exrow, dspm.at[pl.ds(r0 + t * C1, C1)])
    plsc.subcore_barrier()

    @pl.loop(0, EW1 // C1)
    def _chunk(chunk):
        e0 = pl.multiple_of(wid * EW1 + chunk * C1, C1)
        j0 = pl.multiple_of(e0 // 128, C1J)
        pltpu.sync_copy(src_hbm.at[pl.ds(j0, C1J)], srcv)
        pltpu.sync_copy(dst_hbm.at[pl.ds(j0, C1J)], dstv)
        pltpu.sync_copy(et_hbm.at[pl.ds(j0, C1J)], etv)
        for j in range(C1J):
            for k in range(128 // LANES):
                sl = pl.ds(k * LANES, LANES)
                et_n = etv[j, sl] * N
                idxq[j, sl] = et_n + dstv[j, sl]
                idxk[j, sl] = et_n + srcv[j, sl]
        cps = []
        for j in range(C1J):
            cps.append(pltpu.async_copy(
                qk_hbm.at[idxq.at[j]], qrows.at[pl.ds(j * 128, 128)], sem))
            cps.append(pltpu.async_copy(
                qk_hbm.at[idxk.at[j]], krows.at[pl.ds(j * 128, 128)], sem))
        for cp in cps:
            cp.wait()

        @pl.loop(0, C1 // 2)
        def _pair(p):
            ea = pl.multiple_of(2 * p, 2)
            eb = ea + 1
            s_a = qrows[ea] + _splat(krows[ea], sel4)
            s_b = qrows[eb] + _splat(krows[eb], sel4)
            row_a = jnp.where(m4, jnp.exp(jnp.where(s_a >= 0, s_a, 0.2 * s_a)),
                              degc)
            row_b = jnp.where(m4, jnp.exp(jnp.where(s_b >= 0, s_b, 0.2 * s_b)),
                              degc)
            exrow[ea] = row_a
            exrow[eb] = row_b
            bufx[p] = jnp.where(m8, row_a, _splat(row_b, low8))

        for j in range(C1J):
            pltpu.sync_copy(exrow.at[pl.ds(j * 128, 128)],
                            dspm.at[dstv.at[j]], add=True)
        pltpu.sync_copy(
            bufx, ex_hbm.at[pl.ds(pl.multiple_of(e0 // 2, C1 // 2), C1 // 2)])

    plsc.subcore_barrier()
    pltpu.sync_copy(dspm.at[pl.ds(r0, STRIPE)],
                    dp_hbm.at[pl.ds(pl.multiple_of(cid * N + r0, STRIPE),
                                    STRIPE)])


_sc_pass1 = functools.partial(
    pl.kernel,
    out_type=[
        jax.ShapeDtypeStruct((E // 2, LANES), jnp.float32),
        jax.ShapeDtypeStruct((2 * N, 16), jnp.float32),
    ],
    mesh=_mesh,
    scratch_types=[
        pltpu.VMEM((C1J, 128), jnp.int32),
        pltpu.VMEM((C1J, 128), jnp.int32),
        pltpu.VMEM((C1J, 128), jnp.int32),
        pltpu.VMEM((C1J, 128), jnp.int32),
        pltpu.VMEM((C1J, 128), jnp.int32),
        pltpu.VMEM((C1, LANES), jnp.float32),
        pltpu.VMEM((C1, LANES), jnp.float32),
        pltpu.VMEM((C1, LANES), jnp.float32),
        pltpu.VMEM((C1 // 2, LANES), jnp.float32),
        pltpu.VMEM_SHARED((N, 16), jnp.float32),
        pltpu.SemaphoreType.DMA,
    ],
    compiler_params=pltpu.CompilerParams(use_tc_tiling_on_sc=False),
)(_p1_body)


def _make_p2_body(hbase):
    def _p2_body(xr_hbm, src_hbm, dst_hbm, et_hbm, ex_hbm, s_hbm,
                 srcv, dstv, etv, idxv, exc, xrows, msg, sspm, sem):
        cid = lax.axis_index("c")
        sid = lax.axis_index("s")
        zero16 = jnp.zeros((LANES,), jnp.float32)

        @pl.loop(0, C2)
        def _zero(i):
            msg[i] = zero16

        r0 = pl.multiple_of(sid * STRIPE, STRIPE)
        for t in range(STRIPE // C2):
            pltpu.sync_copy(msg, sspm.at[pl.ds(r0 + t * C2, C2)])
        plsc.subcore_barrier()

        base_t = cid * (R * N)
        hsel_a = jnp.full((LANES, 1), hbase + cid, jnp.int32)
        hsel_b = jnp.full((LANES, 1), hbase + cid + 8, jnp.int32)

        @pl.loop(0, EW2 // C2)
        def _chunk(chunk):
            e0 = pl.multiple_of(sid * EW2 + chunk * C2, C2)
            j0 = pl.multiple_of(e0 // 128, C2J)
            pltpu.sync_copy(src_hbm.at[pl.ds(j0, C2J)], srcv)
            pltpu.sync_copy(dst_hbm.at[pl.ds(j0, C2J)], dstv)
            pltpu.sync_copy(et_hbm.at[pl.ds(j0, C2J)], etv)
            pltpu.sync_copy(
                ex_hbm.at[pl.ds(pl.multiple_of(e0 // 2, C2 // 2), C2 // 2)],
                exc)
            for j in range(C2J):
                for k in range(128 // LANES):
                    sl = pl.ds(k * LANES, LANES)
                    idxv[j, sl] = base_t + etv[j, sl] * N + srcv[j, sl]
            cps = []
            for j in range(C2J):
                cps.append(pltpu.async_copy(
                    xr_hbm.at[idxv.at[j]], xrows.at[pl.ds(j * 128, 128)],
                    sem))
            for cp in cps:
                cp.wait()

            @pl.loop(0, C2 // 2)
            def _pair(p):
                ea = pl.multiple_of(2 * p, 2)
                eb = ea + 1
                exv = exc[p]
                msg[ea] = _splat(exv, hsel_a) * xrows[ea]
                msg[eb] = _splat(exv, hsel_b) * xrows[eb]

            for j in range(C2J):
                pltpu.sync_copy(msg.at[pl.ds(j * 128, 128)],
                                sspm.at[dstv.at[j]], add=True)

        plsc.subcore_barrier()
        pltpu.sync_copy(sspm.at[pl.ds(r0, STRIPE)],
                        s_hbm.at[pl.ds(pl.multiple_of(cid * N + r0, STRIPE),
                                       STRIPE)])

    return _p2_body


def _make_p2(hbase):
    return functools.partial(
        pl.kernel,
        out_type=jax.ShapeDtypeStruct((2 * N, 16), jnp.float32),
        mesh=_mesh,
        scratch_types=[
            pltpu.VMEM((C2J, 128), jnp.int32),
            pltpu.VMEM((C2J, 128), jnp.int32),
            pltpu.VMEM((C2J, 128), jnp.int32),
            pltpu.VMEM((C2J, 128), jnp.int32),
            pltpu.VMEM((C2 // 2, LANES), jnp.float32),
            pltpu.VMEM((C2, LANES), jnp.float32),
            pltpu.VMEM((C2, LANES), jnp.float32),
            pltpu.VMEM_SHARED((N, 16), jnp.float32),
            pltpu.SemaphoreType.DMA,
        ],
        compiler_params=pltpu.CompilerParams(use_tc_tiling_on_sc=False),
    )(_make_p2_body(hbase))


_sc_pass2_a = _make_p2(0)
_sc_pass2_b = _make_p2(2)


def _final_body(s00, s01, s02, s03, s10, s11, s12, s13, s20, s21, s22, s23,
                d0l, d0h, d1l, d1h, d2l, d2h, Wi_ref, bi_ref, out_ref):
    s = [[s00, s01, s02, s03], [s10, s11, s12, s13], [s20, s21, s22, s23]]
    d = [(d0l, d0h), (d1l, d1h), (d2l, d2h)]
    feats = [_norm_x(s[v], *d[v]) for v in range(3)]
    feat = jnp.concatenate(feats, axis=1)
    out_ref[...] = (jnp.dot(feat, Wi_ref[...],
                            preferred_element_type=jnp.float32)
                    + bi_ref[...])


def _final(SA, SB, dp, Wi, bi):
    BPV = 2048
    kv = NPV // BPV
    kn = N // BPV

    def s_spec(v, h):
        off = (h % 2) * kn + v * kv
        return pl.BlockSpec((BPV, 16), lambda i, _o=off: (i + _o, 0))

    def d_spec(v, half):
        off = half * kn + v * kv
        return pl.BlockSpec((BPV, 16), lambda i, _o=off: (i + _o, 0))

    in_specs = ([s_spec(v, h) for v in range(3) for h in range(4)]
                + [d_spec(v, hf) for v in range(3) for hf in range(2)]
                + [pl.BlockSpec((3 * HO, NL), lambda i: (0, 0)),
                   pl.BlockSpec((NL,), lambda i: (0,))])
    s_args = [[SA, SA, SB, SB][h] for v in range(3) for h in range(4)]
    d_args = [dp for _v in range(3) for _hf in range(2)]
    return pl.pallas_call(
        _final_body,
        grid=(kv,),
        in_specs=in_specs,
        out_specs=pl.BlockSpec((BPV, NL), lambda i: (i, 0)),
        out_shape=jax.ShapeDtypeStruct((NPV, NL), jnp.float32),
    )(*s_args, *d_args, Wi, bi)


def kernel(x0, x1, x2, edge_index, edge_type, P0, b0, P1, b1, P2, b2,
           basis1, comb1, Q1, K1, basis2, comb2, Q2, K2, Wi, bi):
    src2d = edge_index[0].reshape(E // 128, 128)
    dst2d = edge_index[1].reshape(E // 128, 128)
    et2d = edge_type.reshape(E // 128, 128)

    x = _project(x0, x1, x2, P0, b0, P1, b1, P2, b2)

    xr1, qk1 = _transform1(x, basis1, comb1, Q1, K1)
    xr1a, xr1b = xr1[:2 * R * N], xr1[2 * R * N:]
    ex1, dp1 = _sc_pass1(qk1, src2d, dst2d, et2d)
    SA1 = _sc_pass2_a(xr1a, src2d, dst2d, et2d, ex1)
    SB1 = _sc_pass2_b(xr1b, src2d, dst2d, et2d, ex1)

    xr2, qk2 = _transform2(SA1, SB1, dp1, basis2, comb2, Q2, K2)
    xr2a, xr2b = xr2[:2 * R * N], xr2[2 * R * N:]
    ex2, dp2 = _sc_pass1(qk2, src2d, dst2d, et2d)
    SA2 = _sc_pass2_a(xr2a, src2d, dst2d, et2d, ex2)
    SB2 = _sc_pass2_b(xr2b, src2d, dst2d, et2d, ex2)

    return _final(SA2, SB2, dp2, Wi, bi)


# speedup vs baseline: 34.0981x; 1.2378x over previous
import functools

import jax
import jax.numpy as jnp
from jax import lax
from jax.experimental import pallas as pl
from jax.experimental.pallas import tpu as pltpu
from jax.experimental.pallas import tpu_sc as plsc

NPV = 16384
N = 3 * NPV
E = 786432
R = 4
H = 4
O = 16
HO = 64
NL = 5

NCORE = 2
NSUB = 16
NW = NCORE * NSUB
LANES = 16

C1 = 1024
C1J = C1 // 128
EW1 = E // NW
C2 = 512
C2J = C2 // 128
EW2 = E // NSUB
STRIPE = N // NSUB

_mesh = plsc.VectorSubcoreMesh(core_axis_name="c", subcore_axis_name="s")


def _proj_body(x0_ref, x1_ref, x2_ref, P0_ref, b0_ref, P1_ref, b1_ref,
               P2_ref, b2_ref, out_ref):
    out_ref[0] = jnp.maximum(
        jnp.dot(x0_ref[...], P0_ref[...], preferred_element_type=jnp.float32)
        + b0_ref[...], 0.0)
    out_ref[1] = jnp.maximum(
        jnp.dot(x1_ref[...], P1_ref[...], preferred_element_type=jnp.float32)
        + b1_ref[...], 0.0)
    out_ref[2] = jnp.maximum(
        jnp.dot(x2_ref[...], P2_ref[...], preferred_element_type=jnp.float32)
        + b2_ref[...], 0.0)


def _project(x0, x1, x2, P0, b0, P1, b1, P2, b2):
    BN = 2048
    out = pl.pallas_call(
        _proj_body,
        grid=(NPV // BN,),
        in_specs=[
            pl.BlockSpec((BN, 512), lambda i: (i, 0)),
            pl.BlockSpec((BN, 256), lambda i: (i, 0)),
            pl.BlockSpec((BN, 128), lambda i: (i, 0)),
            pl.BlockSpec((512, HO), lambda i: (0, 0)),
            pl.BlockSpec((HO,), lambda i: (0,)),
            pl.BlockSpec((256, HO), lambda i: (0, 0)),
            pl.BlockSpec((HO,), lambda i: (0,)),
            pl.BlockSpec((128, HO), lambda i: (0, 0)),
            pl.BlockSpec((HO,), lambda i: (0,)),
        ],
        out_specs=pl.BlockSpec((3, BN, HO), lambda i: (0, i, 0)),
        out_shape=jax.ShapeDtypeStruct((3, NPV, HO), jnp.float32),
    )(x0, x1, x2, P0, b0, P1, b1, P2, b2)
    return out.reshape(N, HO)


def _xform_common(x_blk, basis_ref, comb_ref, Q_ref, K_ref, xr_ref, qk_ref):
    bn = x_blk.shape[0]
    for r in range(R):
        W_r = (comb_ref[r, 0] * basis_ref[0]
               + comb_ref[r, 1] * basis_ref[1])
        xr_r = jnp.dot(x_blk, W_r, preferred_element_type=jnp.float32)
        xr_ref[0, r] = xr_r[:, :2 * O]
        xr_ref[1, r] = xr_r[:, 2 * O:]
        cols = []
        for h in range(H):
            qrow = Q_ref[r, h][None, :]
            cols.append(jnp.sum(xr_r[:, h * O:(h + 1) * O] * qrow,
                                axis=1, keepdims=True))
        for h in range(H):
            krow = K_ref[r, h][None, :]
            cols.append(jnp.sum(xr_r[:, h * O:(h + 1) * O] * krow,
                                axis=1, keepdims=True))
        cols.append(jnp.zeros((bn, 8), jnp.float32))
        qk_ref[r] = jnp.concatenate(cols, axis=1)


def _xform1_body(x_ref, basis_ref, comb_ref, Q_ref, K_ref, xr_ref, qk_ref):
    _xform_common(x_ref[...], basis_ref, comb_ref, Q_ref, K_ref,
                  xr_ref, qk_ref)


def _norm_x(slo_ref, shi_ref, dlo_ref, dhi_ref):
    d = dlo_ref[...] + dhi_ref[...]
    deg = d[:, 4:5]
    slo = slo_ref[...]
    shi = shi_ref[...]
    halves = [slo[:, :O], slo[:, O:], shi[:, :O], shi[:, O:]]
    parts = []
    for h in range(H):
        scale = deg / (d[:, h:h + 1] + 1e-16)
        parts.append(halves[h] * scale)
    return jnp.concatenate(parts, axis=1)


def _xform2_body(slo, shi, dlo_ref, dhi_ref, basis_ref, comb_ref,
                 Q_ref, K_ref, xr_ref, qk_ref):
    x_blk = _norm_x(slo, shi, dlo_ref, dhi_ref)
    _xform_common(x_blk, basis_ref, comb_ref, Q_ref, K_ref, xr_ref, qk_ref)


_XFORM_OUT = [
    jax.ShapeDtypeStruct((2, R, N, 2 * O), jnp.float32),
    jax.ShapeDtypeStruct((R, N, 16), jnp.float32),
]

_W_SPECS = [
    pl.BlockSpec((2, HO, HO), lambda i: (0, 0, 0)),
    pl.BlockSpec((R, 2), lambda i: (0, 0)),
    pl.BlockSpec((R, H, O), lambda i: (0, 0, 0)),
    pl.BlockSpec((R, H, O), lambda i: (0, 0, 0)),
]

_BT = 2048


def _xform_out_specs():
    return [
        pl.BlockSpec((2, R, _BT, 2 * O), lambda i: (0, 0, i, 0)),
        pl.BlockSpec((R, _BT, 16), lambda i: (0, i, 0)),
    ]


def _transform1(x, basis, comb, Q, K):
    xr, qk = pl.pallas_call(
        _xform1_body,
        grid=(N // _BT,),
        in_specs=[pl.BlockSpec((_BT, HO), lambda i: (i, 0))] + _W_SPECS,
        out_specs=_xform_out_specs(),
        out_shape=_XFORM_OUT,
    )(x, basis, comb, Q, K)
    return xr.reshape(2 * R * N, 2 * O), qk.reshape(R * N, 16)


def _transform2(S, dp, basis, comb, Q, K):
    kn = N // _BT
    xr, qk = pl.pallas_call(
        _xform2_body,
        grid=(kn,),
        in_specs=[
            pl.BlockSpec((_BT, 2 * O), lambda i: (i, 0)),
            pl.BlockSpec((_BT, 2 * O), lambda i, _k=kn: (i + _k, 0)),
            pl.BlockSpec((_BT, 16), lambda i: (i, 0)),
            pl.BlockSpec((_BT, 16), lambda i, _k=kn: (i + _k, 0)),
        ] + _W_SPECS,
        out_specs=_xform_out_specs(),
        out_shape=_XFORM_OUT,
    )(S, S, dp, dp, basis, comb, Q, K)
    return xr.reshape(2 * R * N, 2 * O), qk.reshape(R * N, 16)


_GDN = lax.GatherDimensionNumbers(
    offset_dims=(), collapsed_slice_dims=(0,), start_index_map=(0,))


def _splat(vec, sel):
    return lax.gather(vec, sel, _GDN, slice_sizes=(1,),
                      mode=lax.GatherScatterMode.PROMISE_IN_BOUNDS)


def _p1_body(qk_hbm, src_hbm, dst_hbm, et_hbm, ex_hbm, dp_hbm,
             srcv, dstv, etv, idxq, idxk, qrows, krows, exrow, bufx,
             dspm, sem):
    cid = lax.axis_index("c")
    sid = lax.axis_index("s")
    wid = sid * NCORE + cid
    lane = lax.iota(jnp.int32, LANES)
    degc = jnp.where(lane == 4, 1.0, 0.0).astype(jnp.float32)
    m4 = lane < 4
    m8 = lane < 8
    low8 = jnp.maximum(lane - 8, 0).reshape(LANES, 1)
    sel4 = jnp.minimum(lane + 4, LANES - 1).reshape(LANES, 1)

    @pl.loop(0, C1)
    def _zero(i):
        exrow[i] = jnp.zeros((LANES,), jnp.float32)

    r0 = pl.multiple_of(sid * STRIPE, STRIPE)
    for t in range(STRIPE // C1):
        pltpu.sync_copy(exrow, dspm.at[pl.ds(r0 + t * C1, C1)])
    plsc.subcore_barrier()

    @pl.loop(0, EW1 // C1)
    def _chunk(chunk):
        e0 = pl.multiple_of(wid * EW1 + chunk * C1, C1)
        j0 = pl.multiple_of(e0 // 128, C1J)
        pltpu.sync_copy(src_hbm.at[pl.ds(j0, C1J)], srcv)
        pltpu.sync_copy(dst_hbm.at[pl.ds(j0, C1J)], dstv)
        pltpu.sync_copy(et_hbm.at[pl.ds(j0, C1J)], etv)
        for j in range(C1J):
            for k in range(128 // LANES):
                sl = pl.ds(k * LANES, LANES)
                et_n = etv[j, sl] * N
                idxq[j, sl] = et_n + dstv[j, sl]
                idxk[j, sl] = et_n + srcv[j, sl]
        cps = []
        for j in range(C1J):
            cps.append(pltpu.async_copy(
                qk_hbm.at[idxq.at[j]], qrows.at[pl.ds(j * 128, 128)], sem))
            cps.append(pltpu.async_copy(
                qk_hbm.at[idxk.at[j]], krows.at[pl.ds(j * 128, 128)], sem))
        for cp in cps:
            cp.wait()

        @pl.loop(0, C1 // 2)
        def _pair(p):
            ea = pl.multiple_of(2 * p, 2)
            eb = ea + 1
            s_a = qrows[ea] + _splat(krows[ea], sel4)
            s_b = qrows[eb] + _splat(krows[eb], sel4)
            row_a = jnp.where(m4, jnp.exp(jnp.where(s_a >= 0, s_a, 0.2 * s_a)),
                              degc)
            row_b = jnp.where(m4, jnp.exp(jnp.where(s_b >= 0, s_b, 0.2 * s_b)),
                              degc)
            exrow[ea] = row_a
            exrow[eb] = row_b
            bufx[p] = jnp.where(m8, row_a, _splat(row_b, low8))

        for j in range(C1J):
            pltpu.sync_copy(exrow.at[pl.ds(j * 128, 128)],
                            dspm.at[dstv.at[j]], add=True)
        pltpu.sync_copy(
            bufx, ex_hbm.at[pl.ds(pl.multiple_of(e0 // 2, C1 // 2), C1 // 2)])

    plsc.subcore_barrier()
    pltpu.sync_copy(dspm.at[pl.ds(r0, STRIPE)],
                    dp_hbm.at[pl.ds(pl.multiple_of(cid * N + r0, STRIPE),
                                    STRIPE)])


_sc_pass1 = functools.partial(
    pl.kernel,
    out_type=[
        jax.ShapeDtypeStruct((E // 2, LANES), jnp.float32),
        jax.ShapeDtypeStruct((2 * N, 16), jnp.float32),
    ],
    mesh=_mesh,
    scratch_types=[
        pltpu.VMEM((C1J, 128), jnp.int32),
        pltpu.VMEM((C1J, 128), jnp.int32),
        pltpu.VMEM((C1J, 128), jnp.int32),
        pltpu.VMEM((C1J, 128), jnp.int32),
        pltpu.VMEM((C1J, 128), jnp.int32),
        pltpu.VMEM((C1, LANES), jnp.float32),
        pltpu.VMEM((C1, LANES), jnp.float32),
        pltpu.VMEM((C1, LANES), jnp.float32),
        pltpu.VMEM((C1 // 2, LANES), jnp.float32),
        pltpu.VMEM_SHARED((N, 16), jnp.float32),
        pltpu.SemaphoreType.DMA,
    ],
    compiler_params=pltpu.CompilerParams(use_tc_tiling_on_sc=False),
)(_p1_body)


def _p2_body(xr_hbm, src_hbm, dst_hbm, et_hbm, ex_hbm, s_hbm,
             srcv, dstv, etv, idxv, exc, xrows, sspm, sem):
    cid = lax.axis_index("c")
    sid = lax.axis_index("s")
    zero16 = jnp.zeros((LANES,), jnp.float32)
    lo = pl.ds(0, LANES)
    hi = pl.ds(LANES, LANES)

    @pl.loop(0, C2)
    def _zero(i):
        xrows[i, lo] = zero16
        xrows[i, hi] = zero16

    r0 = pl.multiple_of(sid * STRIPE, STRIPE)
    for t in range(STRIPE // C2):
        pltpu.sync_copy(xrows, sspm.at[pl.ds(r0 + t * C2, C2)])
    plsc.subcore_barrier()

    base_t = cid * (R * N)
    h0a = jnp.full((LANES, 1), 2 * cid, jnp.int32)
    h1a = jnp.full((LANES, 1), 2 * cid + 1, jnp.int32)
    h0b = jnp.full((LANES, 1), 2 * cid + 8, jnp.int32)
    h1b = jnp.full((LANES, 1), 2 * cid + 9, jnp.int32)

    @pl.loop(0, EW2 // C2)
    def _chunk(chunk):
        e0 = pl.multiple_of(sid * EW2 + chunk * C2, C2)
        j0 = pl.multiple_of(e0 // 128, C2J)
        pltpu.sync_copy(src_hbm.at[pl.ds(j0, C2J)], srcv)
        pltpu.sync_copy(dst_hbm.at[pl.ds(j0, C2J)], dstv)
        pltpu.sync_copy(et_hbm.at[pl.ds(j0, C2J)], etv)
        pltpu.sync_copy(
            ex_hbm.at[pl.ds(pl.multiple_of(e0 // 2, C2 // 2), C2 // 2)],
            exc)
        for j in range(C2J):
            for k in range(128 // LANES):
                sl = pl.ds(k * LANES, LANES)
                idxv[j, sl] = base_t + etv[j, sl] * N + srcv[j, sl]
        cps = []
        for j in range(C2J):
            cps.append(pltpu.async_copy(
                xr_hbm.at[idxv.at[j]], xrows.at[pl.ds(j * 128, 128)],
                sem))
        for cp in cps:
            cp.wait()

        @pl.loop(0, C2 // 2)
        def _pair(p):
            ea = pl.multiple_of(2 * p, 2)
            eb = ea + 1
            exv = exc[p]
            xrows[ea, lo] = _splat(exv, h0a) * xrows[ea, lo]
            xrows[ea, hi] = _splat(exv, h1a) * xrows[ea, hi]
            xrows[eb, lo] = _splat(exv, h0b) * xrows[eb, lo]
            xrows[eb, hi] = _splat(exv, h1b) * xrows[eb, hi]

        for j in range(C2J):
            pltpu.sync_copy(xrows.at[pl.ds(j * 128, 128)],
                            sspm.at[dstv.at[j]], add=True)

    plsc.subcore_barrier()
    pltpu.sync_copy(sspm.at[pl.ds(r0, STRIPE)],
                    s_hbm.at[pl.ds(pl.multiple_of(cid * N + r0, STRIPE),
                                   STRIPE)])


_sc_pass2 = functools.partial(
    pl.kernel,
    out_type=jax.ShapeDtypeStruct((2 * N, 2 * O), jnp.float32),
    mesh=_mesh,
    scratch_types=[
        pltpu.VMEM((C2J, 128), jnp.int32),
        pltpu.VMEM((C2J, 128), jnp.int32),
        pltpu.VMEM((C2J, 128), jnp.int32),
        pltpu.VMEM((C2J, 128), jnp.int32),
        pltpu.VMEM((C2 // 2, LANES), jnp.float32),
        pltpu.VMEM((C2, 2 * O), jnp.float32),
        pltpu.VMEM_SHARED((N, 2 * O), jnp.float32),
        pltpu.SemaphoreType.DMA,
    ],
    compiler_params=pltpu.CompilerParams(use_tc_tiling_on_sc=False),
)(_p2_body)


def _final_body(s0l, s0h, s1l, s1h, s2l, s2h,
                d0l, d0h, d1l, d1h, d2l, d2h, Wi_ref, bi_ref, out_ref):
    s = [(s0l, s0h), (s1l, s1h), (s2l, s2h)]
    d = [(d0l, d0h), (d1l, d1h), (d2l, d2h)]
    feats = [_norm_x(*s[v], *d[v]) for v in range(3)]
    feat = jnp.concatenate(feats, axis=1)
    out_ref[...] = (jnp.dot(feat, Wi_ref[...],
                            preferred_element_type=jnp.float32)
                    + bi_ref[...])


def _final(S, dp, Wi, bi):
    BPV = 2048
    kv = NPV // BPV
    kn = N // BPV

    def s_spec(v, half):
        off = half * kn + v * kv
        return pl.BlockSpec((BPV, 2 * O), lambda i, _o=off: (i + _o, 0))

    def d_spec(v, half):
        off = half * kn + v * kv
        return pl.BlockSpec((BPV, 16), lambda i, _o=off: (i + _o, 0))

    in_specs = ([s_spec(v, hf) for v in range(3) for hf in range(2)]
                + [d_spec(v, hf) for v in range(3) for hf in range(2)]
                + [pl.BlockSpec((3 * HO, NL), lambda i: (0, 0)),
                   pl.BlockSpec((NL,), lambda i: (0,))])
    s_args = [S for _v in range(3) for _hf in range(2)]
    d_args = [dp for _v in range(3) for _hf in range(2)]
    return pl.pallas_call(
        _final_body,
        grid=(kv,),
        in_specs=in_specs,
        out_specs=pl.BlockSpec((BPV, NL), lambda i: (i, 0)),
        out_shape=jax.ShapeDtypeStruct((NPV, NL), jnp.float32),
    )(*s_args, *d_args, Wi, bi)


def kernel(x0, x1, x2, edge_index, edge_type, P0, b0, P1, b1, P2, b2,
           basis1, comb1, Q1, K1, basis2, comb2, Q2, K2, Wi, bi):
    src2d = edge_index[0].reshape(E // 128, 128)
    dst2d = edge_index[1].reshape(E // 128, 128)
    et2d = edge_type.reshape(E // 128, 128)

    x = _project(x0, x1, x2, P0, b0, P1, b1, P2, b2)

    xr1, qk1 = _transform1(x, basis1, comb1, Q1, K1)
    ex1, dp1 = _sc_pass1(qk1, src2d, dst2d, et2d)
    S1 = _sc_pass2(xr1, src2d, dst2d, et2d, ex1)

    xr2, qk2 = _transform2(S1, dp1, basis2, comb2, Q2, K2)
    ex2, dp2 = _sc_pass1(qk2, src2d, dst2d, et2d)
    S2 = _sc_pass2(xr2, src2d, dst2d, et2d, ex2)

    return _final(S2, dp2, Wi, bi)


# speedup vs baseline: 45.0202x; 1.3203x over previous
import functools

import jax
import jax.numpy as jnp
from jax import lax
from jax.experimental import pallas as pl
from jax.experimental.pallas import tpu as pltpu
from jax.experimental.pallas import tpu_sc as plsc

NPV = 16384
N = 3 * NPV
E = 786432
R = 4
H = 4
O = 16
HO = 64
NL = 5

NCORE = 2
NSUB = 16
NW = NCORE * NSUB
LANES = 16

C1 = 1024
C1J = C1 // 128
EW1 = E // NW
C2 = 512
C2J = C2 // 128
EW2 = E // NSUB
STRIPE = N // NSUB

_mesh = plsc.VectorSubcoreMesh(core_axis_name="c", subcore_axis_name="s")


def _proj_body(x0_ref, x1_ref, x2_ref, P0_ref, b0_ref, P1_ref, b1_ref,
               P2_ref, b2_ref, out_ref):
    out_ref[0] = jnp.maximum(
        jnp.dot(x0_ref[...], P0_ref[...], preferred_element_type=jnp.float32)
        + b0_ref[...], 0.0)
    out_ref[1] = jnp.maximum(
        jnp.dot(x1_ref[...], P1_ref[...], preferred_element_type=jnp.float32)
        + b1_ref[...], 0.0)
    out_ref[2] = jnp.maximum(
        jnp.dot(x2_ref[...], P2_ref[...], preferred_element_type=jnp.float32)
        + b2_ref[...], 0.0)


def _project(x0, x1, x2, P0, b0, P1, b1, P2, b2):
    BN = 2048
    out = pl.pallas_call(
        _proj_body,
        grid=(NPV // BN,),
        in_specs=[
            pl.BlockSpec((BN, 512), lambda i: (i, 0)),
            pl.BlockSpec((BN, 256), lambda i: (i, 0)),
            pl.BlockSpec((BN, 128), lambda i: (i, 0)),
            pl.BlockSpec((512, HO), lambda i: (0, 0)),
            pl.BlockSpec((HO,), lambda i: (0,)),
            pl.BlockSpec((256, HO), lambda i: (0, 0)),
            pl.BlockSpec((HO,), lambda i: (0,)),
            pl.BlockSpec((128, HO), lambda i: (0, 0)),
            pl.BlockSpec((HO,), lambda i: (0,)),
        ],
        out_specs=pl.BlockSpec((3, BN, HO), lambda i: (0, i, 0)),
        out_shape=jax.ShapeDtypeStruct((3, NPV, HO), jnp.float32),
    )(x0, x1, x2, P0, b0, P1, b1, P2, b2)
    return out.reshape(N, HO)


def _xform_common(x_blk, basis_ref, comb_ref, Q_ref, K_ref, xr_ref, qk_ref):
    for r in range(R):
        W_r = (comb_ref[r, 0] * basis_ref[0]
               + comb_ref[r, 1] * basis_ref[1])
        xr_r = jnp.dot(x_blk, W_r, preferred_element_type=jnp.float32)
        xr_ref[0, r] = xr_r[:, :2 * O]
        xr_ref[1, r] = xr_r[:, 2 * O:]
        zcol = jnp.zeros((O, 1), jnp.float32)
        blocks = []
        for h in range(H):
            cols = [zcol] * 16
            cols[h] = Q_ref[r, h][:, None]
            cols[4 + h] = K_ref[r, h][:, None]
            blocks.append(jnp.concatenate(cols, axis=1))
        Wqk = jnp.concatenate(blocks, axis=0)
        qk_ref[r] = jnp.dot(xr_r, Wqk, preferred_element_type=jnp.float32)


def _xform1_body(x_ref, basis_ref, comb_ref, Q_ref, K_ref, xr_ref, qk_ref):
    _xform_common(x_ref[...], basis_ref, comb_ref, Q_ref, K_ref,
                  xr_ref, qk_ref)


def _norm_x(slo_ref, shi_ref, dlo_ref, dhi_ref):
    d = dlo_ref[...] + dhi_ref[...]
    deg = d[:, 4:5]
    slo = slo_ref[...]
    shi = shi_ref[...]
    halves = [slo[:, :O], slo[:, O:], shi[:, :O], shi[:, O:]]
    parts = []
    for h in range(H):
        scale = deg / (d[:, h:h + 1] + 1e-16)
        parts.append(halves[h] * scale)
    return jnp.concatenate(parts, axis=1)


def _xform2_body(slo, shi, dlo_ref, dhi_ref, basis_ref, comb_ref,
                 Q_ref, K_ref, xr_ref, qk_ref):
    x_blk = _norm_x(slo, shi, dlo_ref, dhi_ref)
    _xform_common(x_blk, basis_ref, comb_ref, Q_ref, K_ref, xr_ref, qk_ref)


_XFORM_OUT = [
    jax.ShapeDtypeStruct((2, R, N, 2 * O), jnp.float32),
    jax.ShapeDtypeStruct((R, N, 16), jnp.float32),
]

_W_SPECS = [
    pl.BlockSpec((2, HO, HO), lambda i: (0, 0, 0)),
    pl.BlockSpec((R, 2), lambda i: (0, 0)),
    pl.BlockSpec((R, H, O), lambda i: (0, 0, 0)),
    pl.BlockSpec((R, H, O), lambda i: (0, 0, 0)),
]

_BT = 2048


def _xform_out_specs():
    return [
        pl.BlockSpec((2, R, _BT, 2 * O), lambda i: (0, 0, i, 0)),
        pl.BlockSpec((R, _BT, 16), lambda i: (0, i, 0)),
    ]


def _transform1(x, basis, comb, Q, K):
    xr, qk = pl.pallas_call(
        _xform1_body,
        grid=(N // _BT,),
        in_specs=[pl.BlockSpec((_BT, HO), lambda i: (i, 0))] + _W_SPECS,
        out_specs=_xform_out_specs(),
        out_shape=_XFORM_OUT,
    )(x, basis, comb, Q, K)
    return xr.reshape(2 * R * N, 2 * O), qk.reshape(R * N, 16)


def _transform2(S, dp, basis, comb, Q, K):
    kn = N // _BT
    xr, qk = pl.pallas_call(
        _xform2_body,
        grid=(kn,),
        in_specs=[
            pl.BlockSpec((_BT, 2 * O), lambda i: (i, 0)),
            pl.BlockSpec((_BT, 2 * O), lambda i, _k=kn: (i + _k, 0)),
            pl.BlockSpec((_BT, 16), lambda i: (i, 0)),
            pl.BlockSpec((_BT, 16), lambda i, _k=kn: (i + _k, 0)),
        ] + _W_SPECS,
        out_specs=_xform_out_specs(),
        out_shape=_XFORM_OUT,
    )(S, S, dp, dp, basis, comb, Q, K)
    return xr.reshape(2 * R * N, 2 * O), qk.reshape(R * N, 16)


_GDN = lax.GatherDimensionNumbers(
    offset_dims=(), collapsed_slice_dims=(0,), start_index_map=(0,))


def _splat(vec, sel):
    return lax.gather(vec, sel, _GDN, slice_sizes=(1,),
                      mode=lax.GatherScatterMode.PROMISE_IN_BOUNDS)


def _p1_body(qk_hbm, src_hbm, dst_hbm, et_hbm, ex_hbm, dp_hbm,
             srcv, dstv, etv, idxq, idxk, qrows, krows, exrow, bufx,
             dspm, sem):
    cid = lax.axis_index("c")
    sid = lax.axis_index("s")
    wid = sid * NCORE + cid
    lane = lax.iota(jnp.int32, LANES)
    degc = jnp.where(lane == 4, 1.0, 0.0).astype(jnp.float32)
    m4 = lane < 4
    m8 = lane < 8
    low8 = jnp.maximum(lane - 8, 0).reshape(LANES, 1)
    sel4 = jnp.minimum(lane + 4, LANES - 1).reshape(LANES, 1)

    @pl.loop(0, C1)
    def _zero(i):
        exrow[i] = jnp.zeros((LANES,), jnp.float32)

    r0 = pl.multiple_of(sid * STRIPE, STRIPE)
    for t in range(STRIPE // C1):
        pltpu.sync_copy(exrow, dspm.at[pl.ds(r0 + t * C1, C1)])
    plsc.subcore_barrier()

    @pl.loop(0, EW1 // C1)
    def _chunk(chunk):
        e0 = pl.multiple_of(wid * EW1 + chunk * C1, C1)
        j0 = pl.multiple_of(e0 // 128, C1J)
        pltpu.sync_copy(src_hbm.at[pl.ds(j0, C1J)], srcv)
        pltpu.sync_copy(dst_hbm.at[pl.ds(j0, C1J)], dstv)
        pltpu.sync_copy(et_hbm.at[pl.ds(j0, C1J)], etv)
        for j in range(C1J):
            for k in range(128 // LANES):
                sl = pl.ds(k * LANES, LANES)
                et_n = etv[j, sl] * N
                idxq[j, sl] = et_n + dstv[j, sl]
                idxk[j, sl] = et_n + srcv[j, sl]
        cps = []
        for j in range(C1J):
            cps.append(pltpu.async_copy(
                qk_hbm.at[idxq.at[j]], qrows.at[pl.ds(j * 128, 128)], sem))
            cps.append(pltpu.async_copy(
                qk_hbm.at[idxk.at[j]], krows.at[pl.ds(j * 128, 128)], sem))
        for cp in cps:
            cp.wait()

        @pl.loop(0, C1 // 2)
        def _pair(p):
            ea = pl.multiple_of(2 * p, 2)
            eb = ea + 1
            s_a = qrows[ea] + _splat(krows[ea], sel4)
            s_b = qrows[eb] + _splat(krows[eb], sel4)
            row_a = jnp.where(m4, jnp.exp(jnp.where(s_a >= 0, s_a, 0.2 * s_a)),
                              degc)
            row_b = jnp.where(m4, jnp.exp(jnp.where(s_b >= 0, s_b, 0.2 * s_b)),
                              degc)
            exrow[ea] = row_a
            exrow[eb] = row_b
            bufx[p] = jnp.where(m8, row_a, _splat(row_b, low8))

        for j in range(C1J):
            pltpu.sync_copy(exrow.at[pl.ds(j * 128, 128)],
                            dspm.at[dstv.at[j]], add=True)
        pltpu.sync_copy(
            bufx, ex_hbm.at[pl.ds(pl.multiple_of(e0 // 2, C1 // 2), C1 // 2)])

    plsc.subcore_barrier()
    pltpu.sync_copy(dspm.at[pl.ds(r0, STRIPE)],
                    dp_hbm.at[pl.ds(pl.multiple_of(cid * N + r0, STRIPE),
                                    STRIPE)])


_sc_pass1 = functools.partial(
    pl.kernel,
    out_type=[
        jax.ShapeDtypeStruct((E // 2, LANES), jnp.float32),
        jax.ShapeDtypeStruct((2 * N, 16), jnp.float32),
    ],
    mesh=_mesh,
    scratch_types=[
        pltpu.VMEM((C1J, 128), jnp.int32),
        pltpu.VMEM((C1J, 128), jnp.int32),
        pltpu.VMEM((C1J, 128), jnp.int32),
        pltpu.VMEM((C1J, 128), jnp.int32),
        pltpu.VMEM((C1J, 128), jnp.int32),
        pltpu.VMEM((C1, LANES), jnp.float32),
        pltpu.VMEM((C1, LANES), jnp.float32),
        pltpu.VMEM((C1, LANES), jnp.float32),
        pltpu.VMEM((C1 // 2, LANES), jnp.float32),
        pltpu.VMEM_SHARED((N, 16), jnp.float32),
        pltpu.SemaphoreType.DMA,
    ],
    compiler_params=pltpu.CompilerParams(use_tc_tiling_on_sc=False),
)(_p1_body)


def _p2_body(xr_hbm, src_hbm, dst_hbm, et_hbm, ex_hbm, s_hbm,
             srcv, dstv, etv, idxv, exc, xrows, sspm, sem):
    cid = lax.axis_index("c")
    sid = lax.axis_index("s")
    zero16 = jnp.zeros((LANES,), jnp.float32)
    lo = pl.ds(0, LANES)
    hi = pl.ds(LANES, LANES)

    @pl.loop(0, C2)
    def _zero(i):
        xrows[i, lo] = zero16
        xrows[i, hi] = zero16

    r0 = pl.multiple_of(sid * STRIPE, STRIPE)
    for t in range(STRIPE // C2):
        pltpu.sync_copy(xrows, sspm.at[pl.ds(r0 + t * C2, C2)])
    plsc.subcore_barrier()

    base_t = cid * (R * N)
    h0a = jnp.full((LANES, 1), 2 * cid, jnp.int32)
    h1a = jnp.full((LANES, 1), 2 * cid + 1, jnp.int32)
    h0b = jnp.full((LANES, 1), 2 * cid + 8, jnp.int32)
    h1b = jnp.full((LANES, 1), 2 * cid + 9, jnp.int32)

    @pl.loop(0, EW2 // C2)
    def _chunk(chunk):
        e0 = pl.multiple_of(sid * EW2 + chunk * C2, C2)
        j0 = pl.multiple_of(e0 // 128, C2J)
        pltpu.sync_copy(src_hbm.at[pl.ds(j0, C2J)], srcv)
        pltpu.sync_copy(dst_hbm.at[pl.ds(j0, C2J)], dstv)
        pltpu.sync_copy(et_hbm.at[pl.ds(j0, C2J)], etv)
        pltpu.sync_copy(
            ex_hbm.at[pl.ds(pl.multiple_of(e0 // 2, C2 // 2), C2 // 2)],
            exc)
        for j in range(C2J):
            for k in range(128 // LANES):
                sl = pl.ds(k * LANES, LANES)
                idxv[j, sl] = base_t + etv[j, sl] * N + srcv[j, sl]
        cps = []
        for j in range(C2J):
            cps.append(pltpu.async_copy(
                xr_hbm.at[idxv.at[j]], xrows.at[pl.ds(j * 128, 128)],
                sem))
        for cp in cps:
            cp.wait()

        @pl.loop(0, C2 // 2)
        def _pair(p):
            ea = pl.multiple_of(2 * p, 2)
            eb = ea + 1
            exv = exc[p]
            xrows[ea, lo] = _splat(exv, h0a) * xrows[ea, lo]
            xrows[ea, hi] = _splat(exv, h1a) * xrows[ea, hi]
            xrows[eb, lo] = _splat(exv, h0b) * xrows[eb, lo]
            xrows[eb, hi] = _splat(exv, h1b) * xrows[eb, hi]

        for j in range(C2J):
            pltpu.sync_copy(xrows.at[pl.ds(j * 128, 128)],
                            sspm.at[dstv.at[j]], add=True)

    plsc.subcore_barrier()
    pltpu.sync_copy(sspm.at[pl.ds(r0, STRIPE)],
                    s_hbm.at[pl.ds(pl.multiple_of(cid * N + r0, STRIPE),
                                   STRIPE)])


_sc_pass2 = functools.partial(
    pl.kernel,
    out_type=jax.ShapeDtypeStruct((2 * N, 2 * O), jnp.float32),
    mesh=_mesh,
    scratch_types=[
        pltpu.VMEM((C2J, 128), jnp.int32),
        pltpu.VMEM((C2J, 128), jnp.int32),
        pltpu.VMEM((C2J, 128), jnp.int32),
        pltpu.VMEM((C2J, 128), jnp.int32),
        pltpu.VMEM((C2 // 2, LANES), jnp.float32),
        pltpu.VMEM((C2, 2 * O), jnp.float32),
        pltpu.VMEM_SHARED((N, 2 * O), jnp.float32),
        pltpu.SemaphoreType.DMA,
    ],
    compiler_params=pltpu.CompilerParams(use_tc_tiling_on_sc=False),
)(_p2_body)


def _final_body(s0l, s0h, s1l, s1h, s2l, s2h,
                d0l, d0h, d1l, d1h, d2l, d2h, Wi_ref, bi_ref, out_ref):
    s = [(s0l, s0h), (s1l, s1h), (s2l, s2h)]
    d = [(d0l, d0h), (d1l, d1h), (d2l, d2h)]
    feats = [_norm_x(*s[v], *d[v]) for v in range(3)]
    feat = jnp.concatenate(feats, axis=1)
    out_ref[...] = (jnp.dot(feat, Wi_ref[...],
                            preferred_element_type=jnp.float32)
                    + bi_ref[...])


def _final(S, dp, Wi, bi):
    BPV = 2048
    kv = NPV // BPV
    kn = N // BPV

    def s_spec(v, half):
        off = half * kn + v * kv
        return pl.BlockSpec((BPV, 2 * O), lambda i, _o=off: (i + _o, 0))

    def d_spec(v, half):
        off = half * kn + v * kv
        return pl.BlockSpec((BPV, 16), lambda i, _o=off: (i + _o, 0))

    in_specs = ([s_spec(v, hf) for v in range(3) for hf in range(2)]
                + [d_spec(v, hf) for v in range(3) for hf in range(2)]
                + [pl.BlockSpec((3 * HO, NL), lambda i: (0, 0)),
                   pl.BlockSpec((NL,), lambda i: (0,))])
    s_args = [S for _v in range(3) for _hf in range(2)]
    d_args = [dp for _v in range(3) for _hf in range(2)]
    return pl.pallas_call(
        _final_body,
        grid=(kv,),
        in_specs=in_specs,
        out_specs=pl.BlockSpec((BPV, NL), lambda i: (i, 0)),
        out_shape=jax.ShapeDtypeStruct((NPV, NL), jnp.float32),
    )(*s_args, *d_args, Wi, bi)


def kernel(x0, x1, x2, edge_index, edge_type, P0, b0, P1, b1, P2, b2,
           basis1, comb1, Q1, K1, basis2, comb2, Q2, K2, Wi, bi):
    src2d = edge_index[0].reshape(E // 128, 128)
    dst2d = edge_index[1].reshape(E // 128, 128)
    et2d = edge_type.reshape(E // 128, 128)

    x = _project(x0, x1, x2, P0, b0, P1, b1, P2, b2)

    xr1, qk1 = _transform1(x, basis1, comb1, Q1, K1)
    ex1, dp1 = _sc_pass1(qk1, src2d, dst2d, et2d)
    S1 = _sc_pass2(xr1, src2d, dst2d, et2d, ex1)

    xr2, qk2 = _transform2(S1, dp1, basis2, comb2, Q2, K2)
    ex2, dp2 = _sc_pass1(qk2, src2d, dst2d, et2d)
    S2 = _sc_pass2(xr2, src2d, dst2d, et2d, ex2)

    return _final(S2, dp2, Wi, bi)


# speedup vs baseline: 45.7250x; 1.0157x over previous
import functools

import jax
import jax.numpy as jnp
from jax import lax
from jax.experimental import pallas as pl
from jax.experimental.pallas import tpu as pltpu
from jax.experimental.pallas import tpu_sc as plsc

NPV = 16384
N = 3 * NPV
E = 786432
R = 4
H = 4
O = 16
HO = 64
NL = 5

NCORE = 2
NSUB = 16
NW = NCORE * NSUB
LANES = 16

C1 = 1024
C1J = C1 // 128
EW1 = E // NW
C2 = 512
C2J = C2 // 128
EW2 = E // NSUB
STRIPE = N // NSUB

_mesh = plsc.VectorSubcoreMesh(core_axis_name="c", subcore_axis_name="s")


def _proj_body(x0_ref, x1_ref, x2_ref, P0_ref, b0_ref, P1_ref, b1_ref,
               P2_ref, b2_ref, out_ref):
    out_ref[0] = jnp.maximum(
        jnp.dot(x0_ref[...], P0_ref[...], preferred_element_type=jnp.float32,
                precision=lax.Precision.HIGHEST)
        + b0_ref[...], 0.0)
    out_ref[1] = jnp.maximum(
        jnp.dot(x1_ref[...], P1_ref[...], preferred_element_type=jnp.float32,
                precision=lax.Precision.HIGHEST)
        + b1_ref[...], 0.0)
    out_ref[2] = jnp.maximum(
        jnp.dot(x2_ref[...], P2_ref[...], preferred_element_type=jnp.float32,
                precision=lax.Precision.HIGHEST)
        + b2_ref[...], 0.0)


def _project(x0, x1, x2, P0, b0, P1, b1, P2, b2):
    BN = 2048
    out = pl.pallas_call(
        _proj_body,
        grid=(NPV // BN,),
        in_specs=[
            pl.BlockSpec((BN, 512), lambda i: (i, 0)),
            pl.BlockSpec((BN, 256), lambda i: (i, 0)),
            pl.BlockSpec((BN, 128), lambda i: (i, 0)),
            pl.BlockSpec((512, HO), lambda i: (0, 0)),
            pl.BlockSpec((HO,), lambda i: (0,)),
            pl.BlockSpec((256, HO), lambda i: (0, 0)),
            pl.BlockSpec((HO,), lambda i: (0,)),
            pl.BlockSpec((128, HO), lambda i: (0, 0)),
            pl.BlockSpec((HO,), lambda i: (0,)),
        ],
        out_specs=pl.BlockSpec((3, BN, HO), lambda i: (0, i, 0)),
        out_shape=jax.ShapeDtypeStruct((3, NPV, HO), jnp.float32),
    )(x0, x1, x2, P0, b0, P1, b1, P2, b2)
    return out.reshape(N, HO)


def _xform_common(x_blk, basis_ref, comb_ref, Q_ref, K_ref, xr_ref, qk_ref):
    for r in range(R):
        W_r = (comb_ref[r, 0] * basis_ref[0]
               + comb_ref[r, 1] * basis_ref[1])
        xr_r = jnp.dot(x_blk, W_r, preferred_element_type=jnp.float32,
                precision=lax.Precision.HIGHEST)
        xr_ref[0, r] = xr_r[:, :2 * O]
        xr_ref[1, r] = xr_r[:, 2 * O:]
        zcol = jnp.zeros((O, 1), jnp.float32)
        blocks = []
        for h in range(H):
            cols = [zcol] * 16
            cols[h] = Q_ref[r, h][:, None]
            cols[4 + h] = K_ref[r, h][:, None]
            blocks.append(jnp.concatenate(cols, axis=1))
        Wqk = jnp.concatenate(blocks, axis=0)
        qk_ref[r] = jnp.dot(xr_r, Wqk, preferred_element_type=jnp.float32,
                precision=lax.Precision.HIGHEST)


def _xform1_body(x_ref, basis_ref, comb_ref, Q_ref, K_ref, xr_ref, qk_ref):
    _xform_common(x_ref[...], basis_ref, comb_ref, Q_ref, K_ref,
                  xr_ref, qk_ref)


def _norm_x(slo_ref, shi_ref, dlo_ref, dhi_ref):
    d = dlo_ref[...] + dhi_ref[...]
    deg = d[:, 4:5]
    slo = slo_ref[...]
    shi = shi_ref[...]
    halves = [slo[:, :O], slo[:, O:], shi[:, :O], shi[:, O:]]
    parts = []
    for h in range(H):
        scale = deg / (d[:, h:h + 1] + 1e-16)
        parts.append(halves[h] * scale)
    return jnp.concatenate(parts, axis=1)


def _xform2_body(slo, shi, dlo_ref, dhi_ref, basis_ref, comb_ref,
                 Q_ref, K_ref, xr_ref, qk_ref):
    x_blk = _norm_x(slo, shi, dlo_ref, dhi_ref)
    _xform_common(x_blk, basis_ref, comb_ref, Q_ref, K_ref, xr_ref, qk_ref)


_XFORM_OUT = [
    jax.ShapeDtypeStruct((2, R, N, 2 * O), jnp.float32),
    jax.ShapeDtypeStruct((R, N, 16), jnp.float32),
]

_W_SPECS = [
    pl.BlockSpec((2, HO, HO), lambda i: (0, 0, 0)),
    pl.BlockSpec((R, 2), lambda i: (0, 0)),
    pl.BlockSpec((R, H, O), lambda i: (0, 0, 0)),
    pl.BlockSpec((R, H, O), lambda i: (0, 0, 0)),
]

_BT = 2048


def _xform_out_specs():
    return [
        pl.BlockSpec((2, R, _BT, 2 * O), lambda i: (0, 0, i, 0)),
        pl.BlockSpec((R, _BT, 16), lambda i: (0, i, 0)),
    ]


def _transform1(x, basis, comb, Q, K):
    xr, qk = pl.pallas_call(
        _xform1_body,
        grid=(N // _BT,),
        in_specs=[pl.BlockSpec((_BT, HO), lambda i: (i, 0))] + _W_SPECS,
        out_specs=_xform_out_specs(),
        out_shape=_XFORM_OUT,
    )(x, basis, comb, Q, K)
    return xr.reshape(2 * R * N, 2 * O), qk.reshape(R * N, 16)


def _transform2(S, dp, basis, comb, Q, K):
    kn = N // _BT
    xr, qk = pl.pallas_call(
        _xform2_body,
        grid=(kn,),
        in_specs=[
            pl.BlockSpec((_BT, 2 * O), lambda i: (i, 0)),
            pl.BlockSpec((_BT, 2 * O), lambda i, _k=kn: (i + _k, 0)),
            pl.BlockSpec((_BT, 16), lambda i: (i, 0)),
            pl.BlockSpec((_BT, 16), lambda i, _k=kn: (i + _k, 0)),
        ] + _W_SPECS,
        out_specs=_xform_out_specs(),
        out_shape=_XFORM_OUT,
    )(S, S, dp, dp, basis, comb, Q, K)
    return xr.reshape(2 * R * N, 2 * O), qk.reshape(R * N, 16)


_GDN = lax.GatherDimensionNumbers(
    offset_dims=(), collapsed_slice_dims=(0,), start_index_map=(0,))


def _splat(vec, sel):
    return lax.gather(vec, sel, _GDN, slice_sizes=(1,),
                      mode=lax.GatherScatterMode.PROMISE_IN_BOUNDS)


def _p1_body(qk_hbm, src_hbm, dst_hbm, et_hbm, ex_hbm, dp_hbm,
             srcv, dstv, etv, idxq, idxk, qrows, krows, exrow, bufx,
             dspm, sem):
    cid = lax.axis_index("c")
    sid = lax.axis_index("s")
    wid = sid * NCORE + cid
    lane = lax.iota(jnp.int32, LANES)
    degc = jnp.where(lane == 4, 1.0, 0.0).astype(jnp.float32)
    m4 = lane < 4
    m8 = lane < 8
    low8 = jnp.maximum(lane - 8, 0).reshape(LANES, 1)
    sel4 = jnp.minimum(lane + 4, LANES - 1).reshape(LANES, 1)

    @pl.loop(0, C1)
    def _zero(i):
        exrow[i] = jnp.zeros((LANES,), jnp.float32)

    r0 = pl.multiple_of(sid * STRIPE, STRIPE)
    for t in range(STRIPE // C1):
        pltpu.sync_copy(exrow, dspm.at[pl.ds(r0 + t * C1, C1)])
    plsc.subcore_barrier()

    @pl.loop(0, EW1 // C1)
    def _chunk(chunk):
        e0 = pl.multiple_of(wid * EW1 + chunk * C1, C1)
        j0 = pl.multiple_of(e0 // 128, C1J)
        lds = [pltpu.async_copy(src_hbm.at[pl.ds(j0, C1J)], srcv, sem),
               pltpu.async_copy(dst_hbm.at[pl.ds(j0, C1J)], dstv, sem),
               pltpu.async_copy(et_hbm.at[pl.ds(j0, C1J)], etv, sem)]
        for cp in lds:
            cp.wait()
        for j in range(C1J):
            for k in range(128 // LANES):
                sl = pl.ds(k * LANES, LANES)
                et_n = etv[j, sl] * N
                idxq[j, sl] = et_n + dstv[j, sl]
                idxk[j, sl] = et_n + srcv[j, sl]
        cps = []
        for j in range(C1J):
            cps.append(pltpu.async_copy(
                qk_hbm.at[idxq.at[j]], qrows.at[pl.ds(j * 128, 128)], sem))
            cps.append(pltpu.async_copy(
                qk_hbm.at[idxk.at[j]], krows.at[pl.ds(j * 128, 128)], sem))
        for cp in cps:
            cp.wait()

        @pl.loop(0, C1 // 2)
        def _pair(p):
            ea = pl.multiple_of(2 * p, 2)
            eb = ea + 1
            s_a = qrows[ea] + _splat(krows[ea], sel4)
            s_b = qrows[eb] + _splat(krows[eb], sel4)
            row_a = jnp.where(m4, jnp.exp(jnp.where(s_a >= 0, s_a, 0.2 * s_a)),
                              degc)
            row_b = jnp.where(m4, jnp.exp(jnp.where(s_b >= 0, s_b, 0.2 * s_b)),
                              degc)
            exrow[ea] = row_a
            exrow[eb] = row_b
            bufx[p] = jnp.where(m8, row_a, _splat(row_b, low8))

        for j in range(C1J):
            pltpu.sync_copy(exrow.at[pl.ds(j * 128, 128)],
                            dspm.at[dstv.at[j]], add=True)
        pltpu.sync_copy(
            bufx, ex_hbm.at[pl.ds(pl.multiple_of(e0 // 2, C1 // 2), C1 // 2)])

    plsc.subcore_barrier()
    pltpu.sync_copy(dspm.at[pl.ds(r0, STRIPE)],
                    dp_hbm.at[pl.ds(pl.multiple_of(cid * N + r0, STRIPE),
                                    STRIPE)])


_sc_pass1 = functools.partial(
    pl.kernel,
    out_type=[
        jax.ShapeDtypeStruct((E // 2, LANES), jnp.float32),
        jax.ShapeDtypeStruct((2 * N, 16), jnp.float32),
    ],
    mesh=_mesh,
    scratch_types=[
        pltpu.VMEM((C1J, 128), jnp.int32),
        pltpu.VMEM((C1J, 128), jnp.int32),
        pltpu.VMEM((C1J, 128), jnp.int32),
        pltpu.VMEM((C1J, 128), jnp.int32),
        pltpu.VMEM((C1J, 128), jnp.int32),
        pltpu.VMEM((C1, LANES), jnp.float32),
        pltpu.VMEM((C1, LANES), jnp.float32),
        pltpu.VMEM((C1, LANES), jnp.float32),
        pltpu.VMEM((C1 // 2, LANES), jnp.float32),
        pltpu.VMEM_SHARED((N, 16), jnp.float32),
        pltpu.SemaphoreType.DMA,
    ],
    compiler_params=pltpu.CompilerParams(use_tc_tiling_on_sc=False),
)(_p1_body)


def _p2_body(xr_hbm, src_hbm, dst_hbm, et_hbm, ex_hbm, s_hbm,
             srcv, dstv, etv, idxv, exc, xrows, sspm, sem):
    cid = lax.axis_index("c")
    sid = lax.axis_index("s")
    zero16 = jnp.zeros((LANES,), jnp.float32)
    lo = pl.ds(0, LANES)
    hi = pl.ds(LANES, LANES)

    @pl.loop(0, C2)
    def _zero(i):
        xrows[i, lo] = zero16
        xrows[i, hi] = zero16

    r0 = pl.multiple_of(sid * STRIPE, STRIPE)
    for t in range(STRIPE // C2):
        pltpu.sync_copy(xrows, sspm.at[pl.ds(r0 + t * C2, C2)])
    plsc.subcore_barrier()

    base_t = cid * (R * N)
    h0a = jnp.full((LANES, 1), 2 * cid, jnp.int32)
    h1a = jnp.full((LANES, 1), 2 * cid + 1, jnp.int32)
    h0b = jnp.full((LANES, 1), 2 * cid + 8, jnp.int32)
    h1b = jnp.full((LANES, 1), 2 * cid + 9, jnp.int32)

    @pl.loop(0, EW2 // C2)
    def _chunk(chunk):
        e0 = pl.multiple_of(sid * EW2 + chunk * C2, C2)
        j0 = pl.multiple_of(e0 // 128, C2J)
        lds = [pltpu.async_copy(src_hbm.at[pl.ds(j0, C2J)], srcv, sem),
               pltpu.async_copy(dst_hbm.at[pl.ds(j0, C2J)], dstv, sem),
               pltpu.async_copy(et_hbm.at[pl.ds(j0, C2J)], etv, sem),
               pltpu.async_copy(
                   ex_hbm.at[pl.ds(pl.multiple_of(e0 // 2, C2 // 2),
                                   C2 // 2)],
                   exc, sem)]
        for cp in lds:
            cp.wait()
        for j in range(C2J):
            for k in range(128 // LANES):
                sl = pl.ds(k * LANES, LANES)
                idxv[j, sl] = base_t + etv[j, sl] * N + srcv[j, sl]
        cps = []
        for j in range(C2J):
            cps.append(pltpu.async_copy(
                xr_hbm.at[idxv.at[j]], xrows.at[pl.ds(j * 128, 128)],
                sem))
        for cp in cps:
            cp.wait()

        @pl.loop(0, C2 // 2)
        def _pair(p):
            ea = pl.multiple_of(2 * p, 2)
            eb = ea + 1
            exv = exc[p]
            xrows[ea, lo] = _splat(exv, h0a) * xrows[ea, lo]
            xrows[ea, hi] = _splat(exv, h1a) * xrows[ea, hi]
            xrows[eb, lo] = _splat(exv, h0b) * xrows[eb, lo]
            xrows[eb, hi] = _splat(exv, h1b) * xrows[eb, hi]

        for j in range(C2J):
            pltpu.sync_copy(xrows.at[pl.ds(j * 128, 128)],
                            sspm.at[dstv.at[j]], add=True)

    plsc.subcore_barrier()
    pltpu.sync_copy(sspm.at[pl.ds(r0, STRIPE)],
                    s_hbm.at[pl.ds(pl.multiple_of(cid * N + r0, STRIPE),
                                   STRIPE)])


_sc_pass2 = functools.partial(
    pl.kernel,
    out_type=jax.ShapeDtypeStruct((2 * N, 2 * O), jnp.float32),
    mesh=_mesh,
    scratch_types=[
        pltpu.VMEM((C2J, 128), jnp.int32),
        pltpu.VMEM((C2J, 128), jnp.int32),
        pltpu.VMEM((C2J, 128), jnp.int32),
        pltpu.VMEM((C2J, 128), jnp.int32),
        pltpu.VMEM((C2 // 2, LANES), jnp.float32),
        pltpu.VMEM((C2, 2 * O), jnp.float32),
        pltpu.VMEM_SHARED((N, 2 * O), jnp.float32),
        pltpu.SemaphoreType.DMA,
    ],
    compiler_params=pltpu.CompilerParams(use_tc_tiling_on_sc=False),
)(_p2_body)


def _final_body(s0l, s0h, s1l, s1h, s2l, s2h,
                d0l, d0h, d1l, d1h, d2l, d2h, Wi_ref, bi_ref, out_ref):
    s = [(s0l, s0h), (s1l, s1h), (s2l, s2h)]
    d = [(d0l, d0h), (d1l, d1h), (d2l, d2h)]
    feats = [_norm_x(*s[v], *d[v]) for v in range(3)]
    feat = jnp.concatenate(feats, axis=1)
    out_ref[...] = (jnp.dot(feat, Wi_ref[...],
                            preferred_element_type=jnp.float32,
                precision=lax.Precision.HIGHEST)
                    + bi_ref[...])


def _final(S, dp, Wi, bi):
    BPV = 2048
    kv = NPV // BPV
    kn = N // BPV

    def s_spec(v, half):
        off = half * kn + v * kv
        return pl.BlockSpec((BPV, 2 * O), lambda i, _o=off: (i + _o, 0))

    def d_spec(v, half):
        off = half * kn + v * kv
        return pl.BlockSpec((BPV, 16), lambda i, _o=off: (i + _o, 0))

    in_specs = ([s_spec(v, hf) for v in range(3) for hf in range(2)]
                + [d_spec(v, hf) for v in range(3) for hf in range(2)]
                + [pl.BlockSpec((3 * HO, NL), lambda i: (0, 0)),
                   pl.BlockSpec((NL,), lambda i: (0,))])
    s_args = [S for _v in range(3) for _hf in range(2)]
    d_args = [dp for _v in range(3) for _hf in range(2)]
    return pl.pallas_call(
        _final_body,
        grid=(kv,),
        in_specs=in_specs,
        out_specs=pl.BlockSpec((BPV, NL), lambda i: (i, 0)),
        out_shape=jax.ShapeDtypeStruct((NPV, NL), jnp.float32),
    )(*s_args, *d_args, Wi, bi)


def kernel(x0, x1, x2, edge_index, edge_type, P0, b0, P1, b1, P2, b2,
           basis1, comb1, Q1, K1, basis2, comb2, Q2, K2, Wi, bi):
    src2d = edge_index[0].reshape(E // 128, 128)
    dst2d = edge_index[1].reshape(E // 128, 128)
    et2d = edge_type.reshape(E // 128, 128)

    x = _project(x0, x1, x2, P0, b0, P1, b1, P2, b2)

    xr1, qk1 = _transform1(x, basis1, comb1, Q1, K1)
    ex1, dp1 = _sc_pass1(qk1, src2d, dst2d, et2d)
    S1 = _sc_pass2(xr1, src2d, dst2d, et2d, ex1)

    xr2, qk2 = _transform2(S1, dp1, basis2, comb2, Q2, K2)
    ex2, dp2 = _sc_pass1(qk2, src2d, dst2d, et2d)
    S2 = _sc_pass2(xr2, src2d, dst2d, et2d, ex2)

    return _final(S2, dp2, Wi, bi)
